# baseline (speedup 1.0000x reference)
import functools

import jax
import jax.numpy as jnp
from jax import lax
from jax.experimental import pallas as pl
from jax.experimental.pallas import tpu as pltpu
from jax.experimental.pallas import tpu_sc as plsc

_N = 16384
_NW = 32
_NPW = _N // _NW
_G = _NPW // 16
_NBINS = 50
_BIN = 20
_PRE = 50
_CHUNK_BINS = 2
_NCHUNKS = _NBINS // _CHUNK_BINS
_ROWS_PER_CHUNK = _CHUNK_BINS * _BIN
_SYNC_COST = 10.0
_TARGET = 0.1

_mesh = plsc.VectorSubcoreMesh(core_axis_name="c", subcore_axis_name="s")


def _bin_counts(buf_ref, slot, j):
    cnt = None
    for g in range(_G):
        acc = buf_ref[slot, _BIN * j, pl.ds(16 * g, 16)]
        for t in range(1, _BIN):
            acc = acc + buf_ref[slot, _BIN * j + t, pl.ds(16 * g, 16)]
        p = plsc.all_reduce_population_count(acc != 0.0)
        cnt = p if cnt is None else cnt + p
    return cnt.astype(jnp.float32)


@functools.partial(
    pl.kernel,
    mesh=_mesh,
    compiler_params=pltpu.CompilerParams(
        use_tc_tiling_on_sc=False, needs_layout_passes=False),
    out_type=jax.ShapeDtypeStruct((_NW, _NBINS, 16), jnp.float32),
    scratch_types=[
        pltpu.VMEM((2, _ROWS_PER_CHUNK, _NPW), jnp.float32),
        pltpu.VMEM((_CHUNK_BINS, 16), jnp.float32),
        pltpu.SemaphoreType.DMA,
        pltpu.SemaphoreType.DMA,
    ],
)
def _sc_counts(x_hbm, out_hbm, buf, cnt, sem0, sem1):
    wid = lax.axis_index("c") * 16 + lax.axis_index("s")
    base = wid * _NPW
    sems = (sem0, sem1)

    def _copy(chunk, slot):
        return pltpu.make_async_copy(
            x_hbm.at[pl.ds(_PRE + _ROWS_PER_CHUNK * chunk, _ROWS_PER_CHUNK),
                     pl.ds(base, _NPW)],
            buf.at[slot],
            sems[slot],
        )

    def _process(chunk, slot):
        _copy(chunk, slot).wait()
        for j in range(_CHUNK_BINS):
            cnt[j, :] = _bin_counts(buf, slot, j)
        pltpu.sync_copy(
            cnt, out_hbm.at[wid, pl.ds(_CHUNK_BINS * chunk, _CHUNK_BINS)])

    _copy(0, 0).start()
    _copy(1, 1).start()

    def _step(i, _):
        c = 2 * i
        _process(c, 0)
        _copy(c + 2, 0).start()
        _process(c + 1, 1)

        @pl.when(c + 3 < _NCHUNKS)
        def _():
            _copy(c + 3, 1).start()

        return _

    lax.fori_loop(0, (_NCHUNKS - 1) // 2, _step, None)
    _process(_NCHUNKS - 1, 0)


def _combine_body(x_ref, out_ref):
    counts = jnp.sum(x_ref[...], axis=0)
    m = jnp.max(counts)
    frac = m / jnp.float32(_N)
    d = frac - jnp.float32(_TARGET)
    out_ref[0, 0] = jnp.float32(_SYNC_COST) * d * d


def kernel(spikes):
    x2d = spikes.reshape(4 * 1100, _N)
    partial_counts = _sc_counts(x2d)
    out = pl.pallas_call(
        _combine_body,
        in_specs=[pl.BlockSpec((_NW, _NBINS, 16), lambda: (0, 0, 0))],
        out_specs=pl.BlockSpec(memory_space=pltpu.SMEM),
        out_shape=jax.ShapeDtypeStruct((1, 1), jnp.float32),
    )(partial_counts)
    return out[0, 0]


# speedup vs baseline: 6.1584x; 6.1584x over previous
import functools

import jax
import jax.numpy as jnp
from jax import lax
from jax.experimental import pallas as pl
from jax.experimental.pallas import tpu as pltpu
from jax.experimental.pallas import tpu_sc as plsc

_N = 16384
_NW = 32
_NPW = _N // _NW
_G = _NPW // 16
_NBINS = 50
_ROWS = 40
_NCHUNKS = 26
_SYNC_COST = 10.0
_TARGET = 0.1

_mesh = plsc.VectorSubcoreMesh(core_axis_name="c", subcore_axis_name="s")


@functools.partial(
    pl.kernel,
    mesh=_mesh,
    compiler_params=pltpu.CompilerParams(needs_layout_passes=False),
    out_type=jax.ShapeDtypeStruct((_NW, _NCHUNKS, 2, 16), jnp.float32),
    scratch_types=[
        pltpu.VMEM((2, _ROWS, _NPW), jnp.float32),
        pltpu.VMEM((_NPW,), jnp.float32),
        pltpu.VMEM((2, 16), jnp.float32),
        pltpu.SemaphoreType.DMA,
        pltpu.SemaphoreType.DMA,
    ],
)
def _sc_counts(x_hbm, out_hbm, buf, pend, cnt, sem0, sem1):
    wid = lax.axis_index("c") * 16 + lax.axis_index("s")
    base = wid * _NPW
    sems = (sem0, sem1)
    zero16 = jnp.zeros((16,), jnp.float32)

    def _copy(chunk, slot):
        return pltpu.make_async_copy(
            x_hbm.at[pl.ds(48 + _ROWS * chunk, _ROWS), pl.ds(base, _NPW)],
            buf.at[slot],
            sems[slot],
        )

    def _process(chunk, slot):
        _copy(chunk, slot).wait()
        cnt0 = None
        cnt1 = None
        for g in range(_G):
            sl = pl.ds(16 * g, 16)
            tot = pend[sl] + buf[slot, 0, sl] + buf[slot, 1, sl]
            p0 = plsc.all_reduce_population_count(tot != 0.0)
            cnt0 = p0 if cnt0 is None else cnt0 + p0
            acc = buf[slot, 2, sl]
            for t in range(3, 22):
                acc = acc + buf[slot, t, sl]
            p1 = plsc.all_reduce_population_count(acc != 0.0)
            cnt1 = p1 if cnt1 is None else cnt1 + p1
            newp = buf[slot, 22, sl]
            for t in range(23, _ROWS):
                newp = newp + buf[slot, t, sl]
            pend[sl] = newp
        cnt[0, :] = jnp.where(chunk > 0, cnt0.astype(jnp.float32), zero16)
        cnt[1, :] = jnp.where(chunk < _NCHUNKS - 1,
                              cnt1.astype(jnp.float32), zero16)
        pltpu.sync_copy(cnt, out_hbm.at[wid, chunk])

    _copy(0, 0).start()
    _copy(1, 1).start()

    def _step(i, carry):
        c = 2 * i
        _process(c, 0)

        @pl.when(c + 2 < _NCHUNKS)
        def _():
            _copy(c + 2, 0).start()

        _process(c + 1, 1)

        @pl.when(c + 3 < _NCHUNKS)
        def _():
            _copy(c + 3, 1).start()

        return carry

    lax.fori_loop(0, _NCHUNKS // 2, _step, None)


def _combine_body(x_ref, out_ref):
    counts = jnp.sum(x_ref[...], axis=0)
    m = jnp.max(counts)
    frac = m / jnp.float32(_N)
    d = frac - jnp.float32(_TARGET)
    out_ref[0, 0] = jnp.float32(_SYNC_COST) * d * d


def kernel(spikes):
    x2d = spikes.reshape(4 * 1100, _N)
    partial_counts = _sc_counts(x2d)
    out = pl.pallas_call(
        _combine_body,
        in_specs=[pl.BlockSpec((_NW, _NCHUNKS, 2, 16), lambda: (0, 0, 0, 0))],
        out_specs=pl.BlockSpec(memory_space=pltpu.SMEM),
        out_shape=jax.ShapeDtypeStruct((1, 1), jnp.float32),
    )(partial_counts)
    return out[0, 0]


# speedup vs baseline: 7.1984x; 1.1689x over previous
import functools

import jax
import jax.numpy as jnp
from jax import lax
from jax.experimental import pallas as pl
from jax.experimental.pallas import tpu as pltpu
from jax.experimental.pallas import tpu_sc as plsc

_N = 16384
_NW = 32
_NPW = _N // _NW
_G = _NPW // 16
_NBINS = 50
_ROWS = 80
_NCHUNKS = 13
_SYNC_COST = 10.0
_TARGET = 0.1

_mesh = plsc.VectorSubcoreMesh(core_axis_name="c", subcore_axis_name="s")


@functools.partial(
    pl.kernel,
    mesh=_mesh,
    compiler_params=pltpu.CompilerParams(needs_layout_passes=False),
    out_type=jax.ShapeDtypeStruct((_NW, _NCHUNKS, 4, 16), jnp.float32),
    scratch_types=[
        pltpu.VMEM((2, _ROWS, _NPW), jnp.float32),
        pltpu.VMEM((_NPW,), jnp.float32),
        pltpu.VMEM((4, 16), jnp.float32),
        pltpu.SemaphoreType.DMA,
        pltpu.SemaphoreType.DMA,
    ],
)
def _sc_counts(x_hbm, out_hbm, buf, pend, cnt, sem0, sem1):
    wid = lax.axis_index("c") * 16 + lax.axis_index("s")
    base = wid * _NPW
    sems = (sem0, sem1)
    zero16 = jnp.zeros((16,), jnp.float32)
    zcnt = jnp.zeros((16,), jnp.int32)

    def _copy(chunk, slot):
        return pltpu.make_async_copy(
            x_hbm.at[pl.ds(48 + _ROWS * chunk, _ROWS), pl.ds(base, _NPW)],
            buf.at[slot],
            sems[slot],
        )

    def _binsum(slot, lo, hi, sl):
        acc = buf[slot, lo, sl]
        for t in range(lo + 1, hi):
            acc = acc + buf[slot, t, sl]
        return acc

    def _process(chunk, slot):
        _copy(chunk, slot).wait()

        def gbody(g, cs):
            ct, ca, cb, cc = cs
            sl = pl.ds(16 * g, 16)
            tot = pend[sl] + buf[slot, 0, sl] + buf[slot, 1, sl]
            ct = ct + plsc.all_reduce_population_count(tot != 0.0)
            ca = ca + plsc.all_reduce_population_count(
                _binsum(slot, 2, 22, sl) != 0.0)
            cb = cb + plsc.all_reduce_population_count(
                _binsum(slot, 22, 42, sl) != 0.0)
            cc = cc + plsc.all_reduce_population_count(
                _binsum(slot, 42, 62, sl) != 0.0)
            pend[sl] = _binsum(slot, 62, _ROWS, sl)
            return (ct, ca, cb, cc)

        ct, ca, cb, cc = lax.fori_loop(0, _G, gbody, (zcnt, zcnt, zcnt, zcnt))
        cnt[0, :] = jnp.where(chunk > 0, ct.astype(jnp.float32), zero16)
        cnt[1, :] = ca.astype(jnp.float32)
        cnt[2, :] = cb.astype(jnp.float32)
        cnt[3, :] = jnp.where(chunk < _NCHUNKS - 1,
                              cc.astype(jnp.float32), zero16)
        pltpu.sync_copy(cnt, out_hbm.at[wid, chunk])

    _copy(0, 0).start()
    _copy(1, 1).start()

    def _step(i, carry):
        c = 2 * i
        _process(c, 0)

        @pl.when(c + 2 < _NCHUNKS)
        def _():
            _copy(c + 2, 0).start()

        _process(c + 1, 1)

        @pl.when(c + 3 < _NCHUNKS)
        def _():
            _copy(c + 3, 1).start()

        return carry

    lax.fori_loop(0, _NCHUNKS // 2, _step, None)
    _process(_NCHUNKS - 1, 0)


def _combine_body(x_ref, out_ref):
    counts = jnp.sum(x_ref[...], axis=0)
    m = jnp.max(counts)
    frac = m / jnp.float32(_N)
    d = frac - jnp.float32(_TARGET)
    out_ref[0, 0] = jnp.float32(_SYNC_COST) * d * d


def kernel(spikes):
    x2d = spikes.reshape(4 * 1100, _N)
    partial_counts = _sc_counts(x2d)
    out = pl.pallas_call(
        _combine_body,
        in_specs=[
            pl.BlockSpec((_NW, _NCHUNKS, 4, 16), lambda: (0, 0, 0, 0))
        ],
        out_specs=pl.BlockSpec(memory_space=pltpu.SMEM),
        out_shape=jax.ShapeDtypeStruct((1, 1), jnp.float32),
    )(partial_counts)
    return out[0, 0]
